# baseline (speedup 1.0000x reference)
import functools

import jax
import jax.numpy as jnp
from jax import lax
from jax.experimental import pallas as pl
from jax.experimental.pallas import tpu as pltpu
from jax.experimental.pallas import tpu_sc as plsc

_NC = 2
_NS = 16
_NW = _NC * _NS
_L = 16

_B = 16384
_D = 32
_BPW = _B // _NW
_CHUNK = 128
_NCHUNK = _BPW // _CHUNK


def _sc_body(eu, ei, ubias, ibias, avg, usr, itm,
             pred_o, u_o, it_o,
             idx_u, idx_i, u_v, it_v, ub_v, ib_v, avg_v, pred_v, sem):
    cid = lax.axis_index("c")
    sid = lax.axis_index("s")
    wid = sid * _NC + cid
    base = wid * _BPW
    jb = wid * _NCHUNK

    pltpu.sync_copy(usr.at[pl.ds(jb, _NCHUNK)], idx_u)
    pltpu.sync_copy(itm.at[pl.ds(jb, _NCHUNK)], idx_i)

    handles = []
    for j in range(_NCHUNK):
        rows = pl.ds(j * _CHUNK, _CHUNK)
        handles.append(pltpu.async_copy(eu.at[idx_u.at[j]], u_v.at[rows], sem))
        handles.append(pltpu.async_copy(ei.at[idx_i.at[j]], it_v.at[rows], sem))
        handles.append(pltpu.async_copy(ubias.at[idx_u.at[j]], ub_v.at[rows], sem))
        handles.append(pltpu.async_copy(ibias.at[idx_i.at[j]], ib_v.at[rows], sem))
    pltpu.sync_copy(avg.at[pl.ds(base, _BPW)], avg_v)
    for h in handles:
        h.wait()

    lane = lax.iota(jnp.int32, _L)
    for g in range(_BPW // _L):
        vals = jnp.zeros((_L,), jnp.float32)
        for k in range(_L):
            r = g * _L + k
            lo = u_v[r, pl.ds(0, _L)] * it_v[r, pl.ds(0, _L)]
            hi = u_v[r, pl.ds(_L, _L)] * it_v[r, pl.ds(_L, _L)]
            s = jnp.sum(lo + hi)
            vals = jnp.where(lane == k, s, vals)
        off = pl.ds(g * _L, _L)
        pred_v[off] = vals * 0.7 + avg_v[off] + ub_v[off] + ib_v[off]

    pltpu.sync_copy(pred_v, pred_o.at[pl.ds(base, _BPW)])
    pltpu.sync_copy(u_v, u_o.at[pl.ds(base, _BPW)])
    pltpu.sync_copy(it_v, it_o.at[pl.ds(base, _BPW)])


_sc_kernel = pl.kernel(
    _sc_body,
    out_type=(
        jax.ShapeDtypeStruct((_B,), jnp.float32),
        jax.ShapeDtypeStruct((_B, _D), jnp.float32),
        jax.ShapeDtypeStruct((_B, _D), jnp.float32),
    ),
    mesh=plsc.VectorSubcoreMesh(core_axis_name="c", subcore_axis_name="s"),
    compiler_params=pltpu.CompilerParams(
        needs_layout_passes=False, use_tc_tiling_on_sc=False),
    scratch_types=[
        pltpu.VMEM((_NCHUNK, _CHUNK), jnp.int32),
        pltpu.VMEM((_NCHUNK, _CHUNK), jnp.int32),
        pltpu.VMEM((_BPW, _D), jnp.float32),
        pltpu.VMEM((_BPW, _D), jnp.float32),
        pltpu.VMEM((_BPW,), jnp.float32),
        pltpu.VMEM((_BPW,), jnp.float32),
        pltpu.VMEM((_BPW,), jnp.float32),
        pltpu.VMEM((_BPW,), jnp.float32),
        pltpu.SemaphoreType.DMA,
    ],
)


def kernel(embed_user_weight, embed_item_weight, user_bias, item_bias,
           average, user, item):
    usr2d = user.reshape(_NW * _NCHUNK, _CHUNK)
    itm2d = item.reshape(_NW * _NCHUNK, _CHUNK)
    pred, u, it = _sc_kernel(embed_user_weight, embed_item_weight,
                             user_bias, item_bias, average, usr2d, itm2d)
    return (pred, u, it)
